# Initial kernel scaffold; baseline (speedup 1.0000x reference)
import jax
import jax.numpy as jnp
from jax.experimental import pallas as pl


def kernel(x, table):
    raise NotImplementedError("write your pallas kernel here")


# speedup vs baseline: 1.4990x; 1.4990x over previous
import functools

import jax
import jax.numpy as jnp
from jax import lax
from jax.experimental import pallas as pl
from jax.experimental.pallas import tpu as pltpu
from jax.experimental.pallas import tpu_sc as plsc

_D = 128
_IDX_MINOR = 128
_GATHERS_PER_CHUNK = 2
_CHUNK_ROWS = _IDX_MINOR * _GATHERS_PER_CHUNK


def _make_sc_lookup(n_rows: int):
    info = plsc.get_sparse_core_info()
    nw = info.num_cores * info.num_subcores
    assert n_rows % (nw * _CHUNK_ROWS) == 0
    rows_per_w = n_rows // nw
    n_chunks = rows_per_w // _CHUNK_ROWS
    n_idx_rows = rows_per_w // _IDX_MINOR
    mesh = plsc.VectorSubcoreMesh(core_axis_name="c", subcore_axis_name="s")

    @functools.partial(
        pl.kernel,
        mesh=mesh,
        out_type=jax.ShapeDtypeStruct((n_rows, _D), jnp.float32),
        scratch_types=[
            pltpu.VMEM((n_idx_rows, _IDX_MINOR), jnp.int32),
            pltpu.VMEM((_CHUNK_ROWS, _D), jnp.float32),
            pltpu.VMEM((_CHUNK_ROWS, _D), jnp.float32),
            pltpu.SemaphoreType.DMA,
            pltpu.SemaphoreType.DMA,
            pltpu.SemaphoreType.DMA,
            pltpu.SemaphoreType.DMA,
        ],
    )
    def lookup(idx_hbm, table_hbm, out_hbm, idx_v, rows0, rows1,
               gsem0, gsem1, osem0, osem1):
        wid = lax.axis_index("s") * info.num_cores + lax.axis_index("c")
        base = wid * rows_per_w
        bufs = (rows0, rows1)
        gsems = (gsem0, gsem1)
        osems = (osem0, osem1)

        pltpu.sync_copy(idx_hbm.at[wid], idx_v)

        gather_descs = {}
        scatter_descs = {}

        def fire_gather(c):
            b = c % 2
            ds = []
            for j in range(_GATHERS_PER_CHUNK):
                ds.append(pltpu.async_copy(
                    table_hbm.at[idx_v.at[c * _GATHERS_PER_CHUNK + j]],
                    bufs[b].at[pl.ds(j * _IDX_MINOR, _IDX_MINOR)],
                    gsems[b]))
            gather_descs[c] = ds

        def fire_scatter(c):
            b = c % 2
            scatter_descs[c] = pltpu.async_copy(
                bufs[b],
                out_hbm.at[pl.ds(base + c * _CHUNK_ROWS, _CHUNK_ROWS)],
                osems[b])

        fire_gather(0)
        for c in range(n_chunks):
            for d in gather_descs.pop(c):
                d.wait()
            fire_scatter(c)
            if c + 1 < n_chunks:
                if c >= 1:
                    scatter_descs.pop(c - 1).wait()
                fire_gather(c + 1)
        scatter_descs.pop(n_chunks - 1).wait()

    def run(idx_flat, table):
        idx3 = idx_flat.reshape(nw, n_idx_rows, _IDX_MINOR)
        return lookup(idx3, table)

    return run


def kernel(x, table):
    b, s = x.shape
    n_rows = b * s
    out = _make_sc_lookup(n_rows)(x.reshape(n_rows), table)
    return out.reshape(b, s, table.shape[1])


# speedup vs baseline: 3.4298x; 2.2880x over previous
import functools

import jax
import jax.numpy as jnp
from jax import lax
from jax.experimental import pallas as pl
from jax.experimental.pallas import tpu as pltpu
from jax.experimental.pallas import tpu_sc as plsc

_D = 128
_CHUNK_B = 16


def _make_sc_lookup(batch: int, seq: int, n_items: int):
    info = plsc.get_sparse_core_info()
    nw = info.num_cores * info.num_subcores
    b_per_w = batch // nw
    rows_per_w = b_per_w * seq
    n_chunks = b_per_w // _CHUNK_B
    chunk_rows = _CHUNK_B * seq
    assert batch % nw == 0 and b_per_w % (2 * _CHUNK_B) == 0
    mesh = plsc.VectorSubcoreMesh(core_axis_name="c", subcore_axis_name="s")

    @functools.partial(
        pl.kernel,
        mesh=mesh,
        out_type=jax.ShapeDtypeStruct((batch, seq, _D), jnp.float32),
        scratch_types=[
            pltpu.VMEM((n_items, _D), jnp.float32),
            pltpu.VMEM((rows_per_w,), jnp.int32),
            pltpu.VMEM((_CHUNK_B, seq, _D), jnp.float32),
            pltpu.VMEM((_CHUNK_B, seq, _D), jnp.float32),
            pltpu.SemaphoreType.DMA,
            pltpu.SemaphoreType.DMA,
        ],
    )
    def lookup(idx_hbm, table_hbm, out_hbm, table_v, idx_v, buf0, buf1,
               osem0, osem1):
        wid = lax.axis_index("s") * info.num_cores + lax.axis_index("c")
        bufs = (buf0, buf1)
        osems = (osem0, osem1)

        pltpu.sync_copy(table_hbm, table_v)
        pltpu.sync_copy(idx_hbm.at[wid], idx_v)

        def assemble_chunk(c, buf):
            row_base = c * chunk_rows

            def per_batch(bb, _):
                ids_base = row_base + bb * seq
                ids_lo = idx_v[pl.ds(ids_base, 16)]
                ids_hi = idx_v[pl.ds(ids_base + seq - 16, 16)]
                for s in range(seq):
                    rid = ids_lo[s] if s < 16 else ids_hi[s - (seq - 16)]
                    for j in range(_D // 16):
                        sl = pl.ds(j * 16, 16)
                        buf[bb, s, sl] = table_v[rid, sl]
                return _

            lax.fori_loop(0, _CHUNK_B, per_batch, 0)

        def fire_scatter(c, b):
            return pltpu.async_copy(
                bufs[b],
                out_hbm.at[pl.ds(wid * b_per_w + c * _CHUNK_B, _CHUNK_B)],
                osems[b])

        def drain_scatter(b):
            pltpu.make_async_copy(
                bufs[b], out_hbm.at[pl.ds(0, _CHUNK_B)], osems[b]).wait()

        def pair_body(t, _):
            for b in range(2):
                c = 2 * t + b

                @pl.when(t > 0)
                def _wait():
                    drain_scatter(b)

                assemble_chunk(c, bufs[b])
                fire_scatter(c, b)
            return _

        lax.fori_loop(0, n_chunks // 2, pair_body, 0)
        drain_scatter(0)
        drain_scatter(1)

    def run(x, table):
        idx2 = x.reshape(nw, rows_per_w)
        return lookup(idx2, table)

    return run


def kernel(x, table):
    b, s = x.shape
    return _make_sc_lookup(b, s, table.shape[0])(x, table)


# speedup vs baseline: 7.3099x; 2.1313x over previous
import functools

import jax
import jax.numpy as jnp
from jax import lax
from jax.experimental import pallas as pl
from jax.experimental.pallas import tpu as pltpu
from jax.experimental.pallas import tpu_sc as plsc

_D = 128
_CHUNK_B = 16


def _make_sc_lookup(batch: int, seq: int, n_items: int):
    info = plsc.get_sparse_core_info()
    nw = info.num_cores * info.num_subcores
    b_per_w = batch // nw
    rows_per_w = b_per_w * seq
    n_chunks = b_per_w // _CHUNK_B
    chunk_rows = _CHUNK_B * seq
    assert batch % nw == 0 and b_per_w % (2 * _CHUNK_B) == 0
    mesh = plsc.VectorSubcoreMesh(core_axis_name="c", subcore_axis_name="s")

    @functools.partial(
        pl.kernel,
        mesh=mesh,
        out_type=jax.ShapeDtypeStruct((batch, seq, _D), jnp.float32),
        scratch_types=[
            pltpu.VMEM_SHARED((n_items, _D), jnp.float32),
            pltpu.VMEM((_CHUNK_B, seq), jnp.int32),
            pltpu.VMEM((_CHUNK_B, seq), jnp.int32),
            pltpu.VMEM((_CHUNK_B, seq, _D), jnp.float32),
            pltpu.VMEM((_CHUNK_B, seq, _D), jnp.float32),
            pltpu.SemaphoreType.DMA,
            pltpu.SemaphoreType.DMA,
            pltpu.SemaphoreType.DMA,
            pltpu.SemaphoreType.DMA,
            pltpu.SemaphoreType.DMA,
            pltpu.SemaphoreType.DMA,
        ],
    )
    def lookup(idx_hbm, table_hbm, out_hbm, table_sh, ibuf0, ibuf1,
               buf0, buf1, isem0, isem1, gsem0, gsem1, osem0, osem1):
        cid = lax.axis_index("c")
        sid = lax.axis_index("s")
        wid = sid * info.num_cores + cid
        ibufs = (ibuf0, ibuf1)
        bufs = (buf0, buf1)
        isems = (isem0, isem1)
        gsems = (gsem0, gsem1)
        osems = (osem0, osem1)

        @pl.when(sid == 0)
        def _stage_table():
            pltpu.sync_copy(table_hbm, table_sh)

        plsc.subcore_barrier()

        idx_descs = {}
        gather_descs = {}
        scatter_descs = {}

        def fire_idx(c):
            b = c % 2
            idx_descs[c] = pltpu.async_copy(
                idx_hbm.at[wid, pl.ds(c * _CHUNK_B, _CHUNK_B)],
                ibufs[b], isems[b])

        def fire_gathers(c):
            b = c % 2
            ds = []
            for k in range(_CHUNK_B):
                ds.append(pltpu.async_copy(
                    table_sh.at[ibufs[b].at[k]],
                    bufs[b].at[k],
                    gsems[b]))
            gather_descs[c] = ds

        def fire_scatter(c):
            b = c % 2
            scatter_descs[c] = pltpu.async_copy(
                bufs[b],
                out_hbm.at[pl.ds(wid * b_per_w + c * _CHUNK_B, _CHUNK_B)],
                osems[b])

        fire_idx(0)
        fire_idx(1)
        idx_descs.pop(0).wait()
        fire_gathers(0)
        for c in range(n_chunks):
            for d in gather_descs.pop(c):
                d.wait()
            fire_scatter(c)
            if c + 2 < n_chunks:
                fire_idx(c + 2)
            if c + 1 < n_chunks:
                if c >= 1:
                    scatter_descs.pop(c - 1).wait()
                idx_descs.pop(c + 1).wait()
                fire_gathers(c + 1)
        scatter_descs.pop(n_chunks - 1).wait()

    def run(x, table):
        idx2 = x.reshape(nw, b_per_w, seq)
        return lookup(idx2, table)

    return run


def kernel(x, table):
    b, s = x.shape
    return _make_sc_lookup(b, s, table.shape[0])(x, table)


# speedup vs baseline: 7.3216x; 1.0016x over previous
import functools

import jax
import jax.numpy as jnp
from jax import lax
from jax.experimental import pallas as pl
from jax.experimental.pallas import tpu as pltpu
from jax.experimental.pallas import tpu_sc as plsc

_D = 128
_CHUNK_B = 16


def _make_sc_lookup(batch: int, seq: int, n_items: int):
    info = plsc.get_sparse_core_info()
    nw = info.num_cores * info.num_subcores
    b_per_w = batch // nw
    rows_per_w = b_per_w * seq
    n_chunks = b_per_w // _CHUNK_B
    chunk_rows = _CHUNK_B * seq
    assert batch % nw == 0 and b_per_w % (2 * _CHUNK_B) == 0
    mesh = plsc.VectorSubcoreMesh(core_axis_name="c", subcore_axis_name="s")

    @functools.partial(
        pl.kernel,
        mesh=mesh,
        compiler_params=pltpu.CompilerParams(use_tc_tiling_on_sc=True),
        out_type=jax.ShapeDtypeStruct((batch, seq, _D), jnp.float32),
        scratch_types=[
            pltpu.VMEM_SHARED((n_items, _D), jnp.float32),
            pltpu.VMEM((_CHUNK_B, seq), jnp.int32),
            pltpu.VMEM((_CHUNK_B, seq), jnp.int32),
            pltpu.VMEM((_CHUNK_B, seq, _D), jnp.float32),
            pltpu.VMEM((_CHUNK_B, seq, _D), jnp.float32),
            pltpu.SemaphoreType.DMA,
            pltpu.SemaphoreType.DMA,
            pltpu.SemaphoreType.DMA,
            pltpu.SemaphoreType.DMA,
            pltpu.SemaphoreType.DMA,
            pltpu.SemaphoreType.DMA,
        ],
    )
    def lookup(idx_hbm, table_hbm, out_hbm, table_sh, ibuf0, ibuf1,
               buf0, buf1, isem0, isem1, gsem0, gsem1, osem0, osem1):
        cid = lax.axis_index("c")
        sid = lax.axis_index("s")
        wid = sid * info.num_cores + cid
        ibufs = (ibuf0, ibuf1)
        bufs = (buf0, buf1)
        isems = (isem0, isem1)
        gsems = (gsem0, gsem1)
        osems = (osem0, osem1)

        @pl.when(sid == 0)
        def _stage_table():
            pltpu.sync_copy(table_hbm, table_sh)

        plsc.subcore_barrier()

        idx_descs = {}
        gather_descs = {}
        scatter_descs = {}

        def fire_idx(c):
            b = c % 2
            idx_descs[c] = pltpu.async_copy(
                idx_hbm.at[wid, pl.ds(c * _CHUNK_B, _CHUNK_B)],
                ibufs[b], isems[b])

        def fire_gathers(c):
            b = c % 2
            ds = []
            for k in range(_CHUNK_B):
                ds.append(pltpu.async_copy(
                    table_sh.at[ibufs[b].at[k]],
                    bufs[b].at[k],
                    gsems[b]))
            gather_descs[c] = ds

        def fire_scatter(c):
            b = c % 2
            scatter_descs[c] = pltpu.async_copy(
                bufs[b],
                out_hbm.at[pl.ds(wid * b_per_w + c * _CHUNK_B, _CHUNK_B)],
                osems[b])

        fire_idx(0)
        fire_idx(1)
        idx_descs.pop(0).wait()
        fire_gathers(0)
        for c in range(n_chunks):
            for d in gather_descs.pop(c):
                d.wait()
            fire_scatter(c)
            if c + 2 < n_chunks:
                fire_idx(c + 2)
            if c + 1 < n_chunks:
                if c >= 1:
                    scatter_descs.pop(c - 1).wait()
                idx_descs.pop(c + 1).wait()
                fire_gathers(c + 1)
        scatter_descs.pop(n_chunks - 1).wait()

    def run(x, table):
        idx2 = x.reshape(nw, b_per_w, seq)
        return lookup(idx2, table)

    return run


def kernel(x, table):
    b, s = x.shape
    return _make_sc_lookup(b, s, table.shape[0])(x, table)
